# baseline (speedup 1.0000x reference)
import functools

import jax
import jax.numpy as jnp
from jax import lax
from jax.experimental import pallas as pl
from jax.experimental.pallas import tpu as pltpu
from jax.experimental.pallas import tpu_sc as plsc

N = 10000
D = 128
H = 8
C = 16
HC = H * C
NCLS = 16

NP = 10240
DUMMY = N
E = 320000
ESL = E + N
NW = 32
K = 32
TE = 10368
EPAD = NW * TE
CHUNKS = TE // K
EXTRA = 2 * K
ROWS_PER_TILE = NP // 16
OW = 144


def _lrelu(v):
    return jnp.maximum(v, 0.2 * v)


def _dot(a, b):
    return jax.lax.dot_general(a, b, (((1,), (0,)), ((), ())),
                               precision=jax.lax.Precision.HIGHEST,
                               preferred_element_type=jnp.float32)


def _tc1_body(x_ref, wl_ref, bl_ref, wr_ref, br_ref, att_ref, m_ref,
              xl_ref, xre_ref):
    x = x_ref[...]
    xl = _dot(x, wl_ref[...]) + bl_ref[...]
    xr = _dot(x, wr_ref[...]) + br_ref[...]
    t = att_ref[...] * _lrelu(xl + xr)
    xl_ref[...] = xl
    xre_ref[...] = jnp.concatenate([xr, _dot(t, m_ref[...])], axis=-1)


def _tc1(x, wl, bl, wr, br, attf, m):
    bn = 512
    grid = (NP // bn,)
    return pl.pallas_call(
        _tc1_body,
        grid=grid,
        in_specs=[
            pl.BlockSpec((bn, D), lambda i: (i, 0)),
            pl.BlockSpec((D, HC), lambda i: (0, 0)),
            pl.BlockSpec((1, HC), lambda i: (0, 0)),
            pl.BlockSpec((D, HC), lambda i: (0, 0)),
            pl.BlockSpec((1, HC), lambda i: (0, 0)),
            pl.BlockSpec((1, HC), lambda i: (0, 0)),
            pl.BlockSpec((HC, C), lambda i: (0, 0)),
        ],
        out_specs=[
            pl.BlockSpec((bn, HC), lambda i: (i, 0)),
            pl.BlockSpec((bn, OW), lambda i: (i, 0)),
        ],
        out_shape=[
            jax.ShapeDtypeStruct((NP, HC), jnp.float32),
            jax.ShapeDtypeStruct((NP, OW), jnp.float32),
        ],
    )(x, wl, bl, wr, br, attf, m)


def _tc2_body(pa_ref, pb_ref, mexp_ref, b_prev_ref, wl_ref, bl_ref,
              wr_ref, br_ref, att_ref, m_ref, xl_ref, xre_ref):
    p = pa_ref[0] + pb_ref[0]
    out_un = p[:, :HC]
    s_exp = _dot(p[:, HC:OW], mexp_ref[...])
    h = jnp.maximum(out_un / s_exp + b_prev_ref[...], 0.0)
    xl = _dot(h, wl_ref[...]) + bl_ref[...]
    xr = _dot(h, wr_ref[...]) + br_ref[...]
    t = att_ref[...] * _lrelu(xl + xr)
    xl_ref[...] = xl
    xre_ref[...] = jnp.concatenate([xr, _dot(t, m_ref[...])], axis=-1)


def _tc2(parts, mexp, b_prev, wl, bl, wr, br, attf, m):
    bn = 512
    grid = (NP // bn,)
    return pl.pallas_call(
        _tc2_body,
        grid=grid,
        in_specs=[
            pl.BlockSpec((1, bn, OW), lambda i: (0, i, 0)),
            pl.BlockSpec((1, bn, OW), lambda i: (1, i, 0)),
            pl.BlockSpec((C, HC), lambda i: (0, 0)),
            pl.BlockSpec((1, HC), lambda i: (0, 0)),
            pl.BlockSpec((D, HC), lambda i: (0, 0)),
            pl.BlockSpec((1, HC), lambda i: (0, 0)),
            pl.BlockSpec((D, HC), lambda i: (0, 0)),
            pl.BlockSpec((1, HC), lambda i: (0, 0)),
            pl.BlockSpec((1, HC), lambda i: (0, 0)),
            pl.BlockSpec((HC, C), lambda i: (0, 0)),
        ],
        out_specs=[
            pl.BlockSpec((bn, HC), lambda i: (i, 0)),
            pl.BlockSpec((bn, OW), lambda i: (i, 0)),
        ],
        out_shape=[
            jax.ShapeDtypeStruct((NP, HC), jnp.float32),
            jax.ShapeDtypeStruct((NP, OW), jnp.float32),
        ],
    )(parts, parts, mexp, b_prev, wl, bl, wr, br, attf, m)


NF = 256
FDUMMY = 255
PACKB = 16384
K2 = 128
TE2 = TE + 2 * K2


def _tc3_body(pa_ref, pb_ref, mexp_ref, b_prev_ref, wc_ref, bc_ref, out_ref):
    p = pa_ref[0] + pb_ref[0]
    out_un = p[:, :HC]
    s_exp = _dot(p[:, HC:OW], mexp_ref[...])
    h = out_un / s_exp + b_prev_ref[...]
    r = lax.broadcasted_iota(jnp.int32, (256, HC), 0)
    vis = jnp.sum(jnp.where(r < 100, h, 0.0), axis=0, keepdims=True) / 100.0
    aud = jnp.sum(jnp.where((r >= 100) & (r < 200), h, 0.0), axis=0,
                  keepdims=True) / 100.0
    tx = jnp.sum(jnp.where(r == 200, h, 0.0), axis=0, keepdims=True)
    avg = (vis + aud + tx) / 3.0
    out_ref[...] = _dot(avg, wc_ref[...]) + bc_ref[...]


def _tc3(parts, mexp, b_prev, wc, bc):
    return pl.pallas_call(
        _tc3_body,
        grid=(1,),
        in_specs=[
            pl.BlockSpec((1, NF, OW), lambda i: (0, 0, 0)),
            pl.BlockSpec((1, NF, OW), lambda i: (1, 0, 0)),
            pl.BlockSpec((C, HC), lambda i: (0, 0)),
            pl.BlockSpec((1, HC), lambda i: (0, 0)),
            pl.BlockSpec((HC, NCLS), lambda i: (0, 0)),
            pl.BlockSpec((1, NCLS), lambda i: (0, 0)),
        ],
        out_specs=pl.BlockSpec((1, NCLS), lambda i: (0, 0)),
        out_shape=jax.ShapeDtypeStruct((1, NCLS), jnp.float32),
    )(parts, parts, mexp, b_prev, wc, bc)


def _edge_compute(e, xj_v, xie_v, row_v, att_rows, lane):
    a_vec = jnp.zeros((16,), jnp.float32)
    xjs = []
    for h in range(H):
        xj = xj_v[e, pl.ds(h * C, 16)]
        xi = xie_v[e, pl.ds(h * C, 16)]
        t = att_rows[h] * _lrelu(xi + xj)
        a_vec = jnp.where(lane == h, jnp.sum(t), a_vec)
        xjs.append(xj)
    shr = xie_v[e, pl.ds(HC, 16)]
    ex_vec = jnp.exp(a_vec - shr)
    row_v[e, pl.ds(HC, 16)] = ex_vec
    for h in range(H):
        row_v[e, pl.ds(h * C, 16)] = xjs[h] * ex_vec[h]


def _zero_rows(row_v, nrows):
    zv = jnp.zeros((16,), jnp.float32)

    def _zero_row(r, _):
        for j in range(OW // 16):
            row_v[r, pl.ds(j * 16, 16)] = zv
        return 0

    lax.fori_loop(0, nrows, _zero_row, 0)


def _sc_body1(xl_hbm, xre_hbm, src_hbm, dst_hbm, att_hbm,
              out_hbm, filt_hbm, cnt_hbm,
              sidx0, didx0, sidx1, didx1, xj0, xie0, xj1, xie1,
              didx_s0, didx_s1, row_v, row_w, att_v, filt_v, c16_v, out_sh,
              sem_i0, sem_i1, sem_g0, sem_g1, sem_s0, sem_s1):
    cid = lax.axis_index("c")
    sid = lax.axis_index("s")
    wid = cid * 16 + sid

    pltpu.sync_copy(att_hbm, att_v)
    _zero_rows(row_v, K)
    _zero_rows(row_w, K)

    dpk = jnp.full((16,), FDUMMY * PACKB + DUMMY, jnp.int32)

    def _fill(i, _):
        filt_v[pl.ds(i * 16, 16)] = dpk
        return 0

    lax.fori_loop(0, TE2 // 16, _fill, 0)

    zbase = sid * ROWS_PER_TILE
    for i in range(ROWS_PER_TILE // K):
        pltpu.sync_copy(row_v, out_sh.at[pl.ds(zbase + i * K, K)])
    plsc.subcore_barrier()

    ebase = wid * TE
    lane = lax.broadcasted_iota(jnp.int32, (16,), 0)
    att_rows = [att_v[pl.ds(h * C, 16)] for h in range(H)]
    bufs = [(sidx0, didx0, xj0, xie0, sem_i0, sem_g0, didx_s0, row_v, sem_s0),
            (sidx1, didx1, xj1, xie1, sem_i1, sem_g1, didx_s1, row_w, sem_s1)]

    pltpu.async_copy(src_hbm.at[pl.ds(ebase, K)], sidx0, sem_i0)
    pltpu.async_copy(dst_hbm.at[pl.ds(ebase, K)], didx0, sem_i0)
    pltpu.async_copy(src_hbm.at[pl.ds(ebase + K, K)], sidx1, sem_i1)
    pltpu.async_copy(dst_hbm.at[pl.ds(ebase + K, K)], didx1, sem_i1)
    pltpu.make_async_copy(src_hbm.at[pl.ds(ebase, K)], sidx0, sem_i0).wait()
    pltpu.make_async_copy(dst_hbm.at[pl.ds(ebase, K)], didx0, sem_i0).wait()
    pltpu.async_copy(xl_hbm.at[sidx0], xj0, sem_g0)
    pltpu.async_copy(xre_hbm.at[didx0], xie0, sem_g0)

    def _pair(k2, cnt):
        for sub in range(2):
            k = 2 * k2 + sub
            sidx, didx, xj_v, xie_v, sem_i, sem_g, didx_s, rbuf, sem_s = bufs[sub]
            osidx, odidx, oxj, oxie, osem_i, osem_g, _, _, _ = bufs[1 - sub]
            pltpu.make_async_copy(xl_hbm.at[sidx], xj_v, sem_g).wait()
            pltpu.make_async_copy(xre_hbm.at[didx], xie_v, sem_g).wait()

            @pl.when(k2 > 0)
            def _drain():
                pltpu.make_async_copy(rbuf, out_sh.at[didx_s], sem_s).wait()

            for g in range(K // 16):
                sv = sidx[pl.ds(g * 16, 16)]
                dv = didx[pl.ds(g * 16, 16)]
                didx_s[pl.ds(g * 16, 16)] = dv
                mask = dv <= 200
                mi = mask.astype(jnp.int32)
                pos = cnt + plsc.cumsum(mi) - 1
                plsc.store_scatter(filt_v, [pos], dv * PACKB + sv, mask=mask)
                cnt = cnt + plsc.all_reduce_population_count(mask)[0]
            cb2 = ebase + (k + 2) * K
            pltpu.async_copy(src_hbm.at[pl.ds(cb2, K)], sidx, sem_i)
            pltpu.async_copy(dst_hbm.at[pl.ds(cb2, K)], didx, sem_i)
            cb1 = ebase + (k + 1) * K
            pltpu.make_async_copy(src_hbm.at[pl.ds(cb1, K)], osidx, osem_i).wait()
            pltpu.make_async_copy(dst_hbm.at[pl.ds(cb1, K)], odidx, osem_i).wait()
            pltpu.async_copy(xl_hbm.at[osidx], oxj, osem_g)
            pltpu.async_copy(xre_hbm.at[odidx], oxie, osem_g)

            def _edge(e2, _):
                _edge_compute(2 * e2, xj_v, xie_v, rbuf, att_rows, lane)
                _edge_compute(2 * e2 + 1, xj_v, xie_v, rbuf, att_rows, lane)
                return 0

            lax.fori_loop(0, K // 2, _edge, 0)
            pltpu.async_copy(rbuf, out_sh.at[didx_s], sem_s, add=True)
        return cnt

    cnt = lax.fori_loop(0, CHUNKS // 2, _pair, jnp.int32(0))

    pltpu.make_async_copy(xl_hbm.at[sidx0], xj0, sem_g0).wait()
    pltpu.make_async_copy(xre_hbm.at[didx0], xie0, sem_g0).wait()
    pltpu.make_async_copy(src_hbm.at[pl.ds(ebase, K)], sidx1, sem_i1).wait()
    pltpu.make_async_copy(dst_hbm.at[pl.ds(ebase, K)], didx1, sem_i1).wait()
    pltpu.make_async_copy(row_v, out_sh.at[didx_s0], sem_s0).wait()
    pltpu.make_async_copy(row_w, out_sh.at[didx_s1], sem_s1).wait()

    pltpu.sync_copy(filt_v, filt_hbm.at[pl.ds(wid * TE2, TE2)])
    @pl.when(wid == NW - 1)
    def _tail():
        pltpu.sync_copy(filt_v.at[pl.ds(TE, 2 * K2)],
                        filt_hbm.at[pl.ds(NW * TE2, 2 * K2)])

    c16_v[...] = jnp.full((16,), 1, jnp.int32) * cnt
    pltpu.sync_copy(c16_v, cnt_hbm.at[wid])

    plsc.subcore_barrier()
    wbase = sid * ROWS_PER_TILE
    pltpu.sync_copy(out_sh.at[pl.ds(wbase, ROWS_PER_TILE)],
                    out_hbm.at[cid, pl.ds(wbase, ROWS_PER_TILE)])


def _sc_body2(xl_hbm, xre_hbm, filt_hbm, cnt_hbm, att_hbm, out_hbm,
              pk0, pk1, sidx0, didx0, sidx1, didx1, xj0, xie0, xj1, xie1,
              row_v, att_v, c16_v, out_sh,
              sem_i0, sem_i1, sem_g0, sem_g1):
    cid = lax.axis_index("c")
    sid = lax.axis_index("s")
    wid = cid * 16 + sid

    pltpu.sync_copy(att_hbm, att_v)
    _zero_rows(row_v, K2)

    pltpu.sync_copy(row_v.at[pl.ds(0, NF // 16)],
                    out_sh.at[pl.ds(sid * (NF // 16), NF // 16)])
    plsc.subcore_barrier()

    pltpu.sync_copy(cnt_hbm.at[wid], c16_v)
    cnt = c16_v[pl.ds(0, 16)][0]
    npair = (cnt + (2 * K2 - 1)) // (2 * K2)

    lane = lax.broadcasted_iota(jnp.int32, (16,), 0)
    att_rows = [att_v[pl.ds(h * C, 16)] for h in range(H)]
    lbase = wid * TE2
    bufs = [(pk0, sidx0, didx0, xj0, xie0, sem_i0, sem_g0),
            (pk1, sidx1, didx1, xj1, xie1, sem_i1, sem_g1)]

    def _unpack(pk, sidx, didx):
        for g in range(K2 // 16):
            p = pk[pl.ds(g * 16, 16)]
            sidx[pl.ds(g * 16, 16)] = jnp.bitwise_and(p, PACKB - 1)
            didx[pl.ds(g * 16, 16)] = lax.shift_right_logical(p, 14)

    pltpu.async_copy(filt_hbm.at[pl.ds(lbase, K2)], pk0, sem_i0)
    pltpu.async_copy(filt_hbm.at[pl.ds(lbase + K2, K2)], pk1, sem_i1)
    pltpu.make_async_copy(filt_hbm.at[pl.ds(lbase, K2)], pk0, sem_i0).wait()
    _unpack(pk0, sidx0, didx0)
    pltpu.async_copy(xl_hbm.at[sidx0], xj0, sem_g0)
    pltpu.async_copy(xre_hbm.at[didx0], xie0, sem_g0)

    def _pair(kp, _):
        for sub in range(2):
            k = 2 * kp + sub
            pk, sidx, didx, xj_v, xie_v, sem_i, sem_g = bufs[sub]
            opk, osidx, odidx, oxj, oxie, osem_i, osem_g = bufs[1 - sub]
            pltpu.make_async_copy(xl_hbm.at[sidx], xj_v, sem_g).wait()
            pltpu.make_async_copy(xre_hbm.at[didx], xie_v, sem_g).wait()
            cb2 = lbase + (k + 2) * K2
            pltpu.async_copy(filt_hbm.at[pl.ds(cb2, K2)], pk, sem_i)
            pltpu.make_async_copy(filt_hbm.at[pl.ds(cb2, K2)], opk, osem_i).wait()
            _unpack(opk, osidx, odidx)
            pltpu.async_copy(xl_hbm.at[osidx], oxj, osem_g)
            pltpu.async_copy(xre_hbm.at[odidx], oxie, osem_g)

            def _edge(e2, _):
                _edge_compute(2 * e2, xj_v, xie_v, row_v, att_rows, lane)
                _edge_compute(2 * e2 + 1, xj_v, xie_v, row_v, att_rows, lane)
                return 0

            lax.fori_loop(0, K2 // 2, _edge, 0)
            pltpu.sync_copy(row_v, out_sh.at[didx], add=True)
        return 0

    lax.fori_loop(0, npair, _pair, 0)

    pltpu.make_async_copy(xl_hbm.at[sidx0], xj0, sem_g0).wait()
    pltpu.make_async_copy(xre_hbm.at[didx0], xie0, sem_g0).wait()
    pltpu.make_async_copy(filt_hbm.at[pl.ds(lbase, K2)], pk1, sem_i1).wait()

    plsc.subcore_barrier()
    wbase = sid * (NF // 16)
    pltpu.sync_copy(out_sh.at[pl.ds(wbase, NF // 16)],
                    out_hbm.at[cid, pl.ds(wbase, NF // 16)])


def _sc_mesh():
    return plsc.VectorSubcoreMesh(core_axis_name="c", subcore_axis_name="s",
                                  num_cores=2, num_subcores=16)


_SC_PARAMS = pltpu.CompilerParams(needs_layout_passes=False,
                                  use_tc_tiling_on_sc=False)


def _sc_edge_pass1(xl, xre, src, dst, attf):
    f = pl.kernel(
        _sc_body1,
        out_type=[
            jax.ShapeDtypeStruct((2, NP, OW), jnp.float32),
            jax.ShapeDtypeStruct((NW * TE2 + 2 * K2,), jnp.int32),
            jax.ShapeDtypeStruct((NW, 16), jnp.int32),
        ],
        mesh=_sc_mesh(),
        compiler_params=_SC_PARAMS,
        scratch_types=[
            pltpu.VMEM((K,), jnp.int32),
            pltpu.VMEM((K,), jnp.int32),
            pltpu.VMEM((K,), jnp.int32),
            pltpu.VMEM((K,), jnp.int32),
            pltpu.VMEM((K, HC), jnp.float32),
            pltpu.VMEM((K, OW), jnp.float32),
            pltpu.VMEM((K, HC), jnp.float32),
            pltpu.VMEM((K, OW), jnp.float32),
            pltpu.VMEM((K,), jnp.int32),
            pltpu.VMEM((K,), jnp.int32),
            pltpu.VMEM((K, OW), jnp.float32),
            pltpu.VMEM((K, OW), jnp.float32),
            pltpu.VMEM((HC,), jnp.float32),
            pltpu.VMEM((TE2,), jnp.int32),
            pltpu.VMEM((16,), jnp.int32),
            pltpu.VMEM_SHARED((NP, OW), jnp.float32),
            pltpu.SemaphoreType.DMA,
            pltpu.SemaphoreType.DMA,
            pltpu.SemaphoreType.DMA,
            pltpu.SemaphoreType.DMA,
            pltpu.SemaphoreType.DMA,
            pltpu.SemaphoreType.DMA,
        ],
    )
    return f(xl, xre, src, dst, attf)


def _sc_edge_pass2(xl, xre, filt, cnts, attf):
    f = pl.kernel(
        _sc_body2,
        out_type=jax.ShapeDtypeStruct((2, NF, OW), jnp.float32),
        mesh=_sc_mesh(),
        compiler_params=_SC_PARAMS,
        scratch_types=[
            pltpu.VMEM((K2,), jnp.int32),
            pltpu.VMEM((K2,), jnp.int32),
            pltpu.VMEM((K2,), jnp.int32),
            pltpu.VMEM((K2,), jnp.int32),
            pltpu.VMEM((K2,), jnp.int32),
            pltpu.VMEM((K2,), jnp.int32),
            pltpu.VMEM((K2, HC), jnp.float32),
            pltpu.VMEM((K2, OW), jnp.float32),
            pltpu.VMEM((K2, HC), jnp.float32),
            pltpu.VMEM((K2, OW), jnp.float32),
            pltpu.VMEM((K2, OW), jnp.float32),
            pltpu.VMEM((HC,), jnp.float32),
            pltpu.VMEM((16,), jnp.int32),
            pltpu.VMEM_SHARED((NF, OW), jnp.float32),
            pltpu.SemaphoreType.DMA,
            pltpu.SemaphoreType.DMA,
            pltpu.SemaphoreType.DMA,
            pltpu.SemaphoreType.DMA,
        ],
    )
    return f(xl, xre, filt, cnts, attf)


def kernel(x, edgeIds, W1l, b1l, W1r, b1r, att1, bias1,
           W2l, b2l, W2r, b2r, att2, bias2, Wc, bc):
    f32 = jnp.float32
    xp = jnp.zeros((NP, D), f32).at[:N].set(x)
    loop = jnp.arange(N, dtype=jnp.int32)

    halfw = NW // 2 * TE
    hpad = jnp.full((halfw - E // 2 - N // 2,), DUMMY, jnp.int32)
    tpad = jnp.full((halfw - E // 2 - N // 2 + EXTRA,), DUMMY, jnp.int32)

    def _stream(a):
        a = a.astype(jnp.int32)
        return jnp.concatenate([a[:E // 2], loop[:N // 2], hpad,
                                a[E // 2:], loop[N // 2:], tpad])

    src = _stream(edgeIds[0])
    dst = _stream(edgeIds[1])

    m = (jnp.arange(HC)[:, None] // C == jnp.arange(C)[None, :]).astype(f32)
    mexp = m.T
    att1f = att1.reshape(1, HC)
    att2f = att2.reshape(1, HC)

    xl1, xre1 = _tc1(xp, W1l, b1l.reshape(1, HC), W1r, b1r.reshape(1, HC),
                     att1f, m)
    parts1, filt, cnts = _sc_edge_pass1(xl1, xre1, src, dst, att1f.reshape(HC))
    xl2, xre2 = _tc2(parts1, mexp, bias1.reshape(1, HC), W2l,
                     b2l.reshape(1, HC), W2r, b2r.reshape(1, HC), att2f, m)
    parts2 = _sc_edge_pass2(xl2, xre2, filt, cnts, att2f.reshape(HC))
    out = _tc3(parts2, mexp, bias2.reshape(1, HC), Wc, bc.reshape(1, NCLS))
    return out


# speedup vs baseline: 1.0974x; 1.0974x over previous
import functools

import jax
import jax.numpy as jnp
from jax import lax
from jax.experimental import pallas as pl
from jax.experimental.pallas import tpu as pltpu
from jax.experimental.pallas import tpu_sc as plsc

N = 10000
D = 128
H = 8
C = 16
HC = H * C
NCLS = 16

NP = 10240
DUMMY = N
E = 320000
ESL = E + N
NW = 32
K = 32
TE = 10368
EPAD = NW * TE
CHUNKS = TE // K
EXTRA = 2 * K
ROWS_PER_TILE = NP // 16
OW = 144


def _lrelu(v):
    return jnp.maximum(v, 0.2 * v)


def _dot(a, b):
    return jax.lax.dot_general(a, b, (((1,), (0,)), ((), ())),
                               precision=jax.lax.Precision.HIGHEST,
                               preferred_element_type=jnp.float32)


def _tc1_body(x_ref, wl_ref, bl_ref, wr_ref, br_ref, att_ref, m_ref,
              xl_ref, xre_ref):
    x = x_ref[...]
    xl = _dot(x, wl_ref[...]) + bl_ref[...]
    xr = _dot(x, wr_ref[...]) + br_ref[...]
    t = att_ref[...] * _lrelu(xl + xr)
    xl_ref[...] = xl
    xre_ref[...] = jnp.concatenate([xr, _dot(t, m_ref[...])], axis=-1)


def _tc1(x, wl, bl, wr, br, attf, m):
    bn = 512
    grid = (NP // bn,)
    return pl.pallas_call(
        _tc1_body,
        grid=grid,
        in_specs=[
            pl.BlockSpec((bn, D), lambda i: (i, 0)),
            pl.BlockSpec((D, HC), lambda i: (0, 0)),
            pl.BlockSpec((1, HC), lambda i: (0, 0)),
            pl.BlockSpec((D, HC), lambda i: (0, 0)),
            pl.BlockSpec((1, HC), lambda i: (0, 0)),
            pl.BlockSpec((1, HC), lambda i: (0, 0)),
            pl.BlockSpec((HC, C), lambda i: (0, 0)),
        ],
        out_specs=[
            pl.BlockSpec((bn, HC), lambda i: (i, 0)),
            pl.BlockSpec((bn, OW), lambda i: (i, 0)),
        ],
        out_shape=[
            jax.ShapeDtypeStruct((NP, HC), jnp.float32),
            jax.ShapeDtypeStruct((NP, OW), jnp.float32),
        ],
    )(x, wl, bl, wr, br, attf, m)


def _tc2_body(pa_ref, pb_ref, mexp_ref, b_prev_ref, wl_ref, bl_ref,
              wr_ref, br_ref, att_ref, m_ref, xl_ref, xre_ref):
    p = pa_ref[0] + pb_ref[0]
    out_un = p[:, :HC]
    s_exp = _dot(p[:, HC:OW], mexp_ref[...])
    h = jnp.maximum(out_un / s_exp + b_prev_ref[...], 0.0)
    xl = _dot(h, wl_ref[...]) + bl_ref[...]
    xr = _dot(h, wr_ref[...]) + br_ref[...]
    t = att_ref[...] * _lrelu(xl + xr)
    xl_ref[...] = xl
    xre_ref[...] = jnp.concatenate([xr, _dot(t, m_ref[...])], axis=-1)


def _tc2(parts, mexp, b_prev, wl, bl, wr, br, attf, m):
    bn = 512
    grid = (NP // bn,)
    return pl.pallas_call(
        _tc2_body,
        grid=grid,
        in_specs=[
            pl.BlockSpec((1, bn, OW), lambda i: (0, i, 0)),
            pl.BlockSpec((1, bn, OW), lambda i: (1, i, 0)),
            pl.BlockSpec((C, HC), lambda i: (0, 0)),
            pl.BlockSpec((1, HC), lambda i: (0, 0)),
            pl.BlockSpec((D, HC), lambda i: (0, 0)),
            pl.BlockSpec((1, HC), lambda i: (0, 0)),
            pl.BlockSpec((D, HC), lambda i: (0, 0)),
            pl.BlockSpec((1, HC), lambda i: (0, 0)),
            pl.BlockSpec((1, HC), lambda i: (0, 0)),
            pl.BlockSpec((HC, C), lambda i: (0, 0)),
        ],
        out_specs=[
            pl.BlockSpec((bn, HC), lambda i: (i, 0)),
            pl.BlockSpec((bn, OW), lambda i: (i, 0)),
        ],
        out_shape=[
            jax.ShapeDtypeStruct((NP, HC), jnp.float32),
            jax.ShapeDtypeStruct((NP, OW), jnp.float32),
        ],
    )(parts, parts, mexp, b_prev, wl, bl, wr, br, attf, m)


NF = 256
FDUMMY = 255
PACKB = 16384
K2 = 128
TE2 = TE + 2 * K2


def _tc3_body(pa_ref, pb_ref, mexp_ref, b_prev_ref, wc_ref, bc_ref, out_ref):
    p = pa_ref[0] + pb_ref[0]
    out_un = p[:, :HC]
    s_exp = _dot(p[:, HC:OW], mexp_ref[...])
    h = out_un / s_exp + b_prev_ref[...]
    r = lax.broadcasted_iota(jnp.int32, (256, HC), 0)
    vis = jnp.sum(jnp.where(r < 100, h, 0.0), axis=0, keepdims=True) / 100.0
    aud = jnp.sum(jnp.where((r >= 100) & (r < 200), h, 0.0), axis=0,
                  keepdims=True) / 100.0
    tx = jnp.sum(jnp.where(r == 200, h, 0.0), axis=0, keepdims=True)
    avg = (vis + aud + tx) / 3.0
    out_ref[...] = _dot(avg, wc_ref[...]) + bc_ref[...]


def _tc3(parts, mexp, b_prev, wc, bc):
    return pl.pallas_call(
        _tc3_body,
        grid=(1,),
        in_specs=[
            pl.BlockSpec((1, NF, OW), lambda i: (0, 0, 0)),
            pl.BlockSpec((1, NF, OW), lambda i: (1, 0, 0)),
            pl.BlockSpec((C, HC), lambda i: (0, 0)),
            pl.BlockSpec((1, HC), lambda i: (0, 0)),
            pl.BlockSpec((HC, NCLS), lambda i: (0, 0)),
            pl.BlockSpec((1, NCLS), lambda i: (0, 0)),
        ],
        out_specs=pl.BlockSpec((1, NCLS), lambda i: (0, 0)),
        out_shape=jax.ShapeDtypeStruct((1, NCLS), jnp.float32),
    )(parts, parts, mexp, b_prev, wc, bc)


def _edge_compute(e, xj_v, xie_v, row_v, att_rows, lane):
    a_vec = jnp.zeros((16,), jnp.float32)
    xjs = []
    for h in range(H):
        xj = xj_v[e, pl.ds(h * C, 16)]
        xi = xie_v[e, pl.ds(h * C, 16)]
        t = att_rows[h] * _lrelu(xi + xj)
        a_vec = jnp.where(lane == h, jnp.sum(t), a_vec)
        xjs.append(xj)
    shr = xie_v[e, pl.ds(HC, 16)]
    ex_vec = jnp.exp(a_vec - shr)
    row_v[e, pl.ds(HC, 16)] = ex_vec
    for h in range(H):
        row_v[e, pl.ds(h * C, 16)] = xjs[h] * ex_vec[h]


def _zero_rows(row_v, nrows):
    zv = jnp.zeros((16,), jnp.float32)

    def _zero_row(r, _):
        for j in range(OW // 16):
            row_v[r, pl.ds(j * 16, 16)] = zv
        return 0

    lax.fori_loop(0, nrows, _zero_row, 0)


def _sc_body1(xl_hbm, xre_hbm, src_hbm, dst_hbm, att_hbm,
              out_hbm, filt_hbm, cnt_hbm,
              sidx0, didx0, sidx1, didx1, xj0, xie0, xj1, xie1,
              didx_s0, didx_s1, row_v, row_w, att_v, filt_v, c16_v, out_sh,
              sem_i0, sem_i1, sem_g0, sem_g1, sem_s0, sem_s1):
    cid = lax.axis_index("c")
    sid = lax.axis_index("s")
    wid = cid * 16 + sid

    pltpu.sync_copy(att_hbm, att_v)
    _zero_rows(row_v, K)
    _zero_rows(row_w, K)

    dlane = lax.broadcasted_iota(jnp.int32, (16,), 0)

    def _fill(i, _):
        drow = 201 + lax.rem(i, 3) * 16 + dlane
        filt_v[pl.ds(i * 16, 16)] = drow * PACKB + DUMMY
        return 0

    lax.fori_loop(0, TE2 // 16, _fill, 0)

    zbase = sid * ROWS_PER_TILE
    for i in range(ROWS_PER_TILE // K):
        pltpu.sync_copy(row_v, out_sh.at[pl.ds(zbase + i * K, K)])
    plsc.subcore_barrier()

    ebase = wid * TE
    lane = lax.broadcasted_iota(jnp.int32, (16,), 0)
    att_rows = [att_v[pl.ds(h * C, 16)] for h in range(H)]
    bufs = [(sidx0, didx0, xj0, xie0, sem_i0, sem_g0, didx_s0, row_v, sem_s0),
            (sidx1, didx1, xj1, xie1, sem_i1, sem_g1, didx_s1, row_w, sem_s1)]

    pltpu.async_copy(src_hbm.at[pl.ds(ebase, K)], sidx0, sem_i0)
    pltpu.async_copy(dst_hbm.at[pl.ds(ebase, K)], didx0, sem_i0)
    pltpu.async_copy(src_hbm.at[pl.ds(ebase + K, K)], sidx1, sem_i1)
    pltpu.async_copy(dst_hbm.at[pl.ds(ebase + K, K)], didx1, sem_i1)
    pltpu.make_async_copy(src_hbm.at[pl.ds(ebase, K)], sidx0, sem_i0).wait()
    pltpu.make_async_copy(dst_hbm.at[pl.ds(ebase, K)], didx0, sem_i0).wait()
    pltpu.async_copy(xl_hbm.at[sidx0], xj0, sem_g0)
    pltpu.async_copy(xre_hbm.at[didx0], xie0, sem_g0)

    def _pair(k2, cnt):
        for sub in range(2):
            k = 2 * k2 + sub
            sidx, didx, xj_v, xie_v, sem_i, sem_g, didx_s, rbuf, sem_s = bufs[sub]
            osidx, odidx, oxj, oxie, osem_i, osem_g, _, _, _ = bufs[1 - sub]
            pltpu.make_async_copy(xl_hbm.at[sidx], xj_v, sem_g).wait()
            pltpu.make_async_copy(xre_hbm.at[didx], xie_v, sem_g).wait()

            @pl.when(k2 > 0)
            def _drain():
                pltpu.make_async_copy(rbuf, out_sh.at[didx_s], sem_s).wait()

            for g in range(K // 16):
                sv = sidx[pl.ds(g * 16, 16)]
                dv = didx[pl.ds(g * 16, 16)]
                didx_s[pl.ds(g * 16, 16)] = dv
                mask = dv <= 200
                mi = mask.astype(jnp.int32)
                pos = cnt + plsc.cumsum(mi) - 1
                plsc.store_scatter(filt_v, [pos], dv * PACKB + sv, mask=mask)
                cnt = cnt + plsc.all_reduce_population_count(mask)[0]
            cb2 = ebase + (k + 2) * K
            pltpu.async_copy(src_hbm.at[pl.ds(cb2, K)], sidx, sem_i)
            pltpu.async_copy(dst_hbm.at[pl.ds(cb2, K)], didx, sem_i)
            cb1 = ebase + (k + 1) * K
            pltpu.make_async_copy(src_hbm.at[pl.ds(cb1, K)], osidx, osem_i).wait()
            pltpu.make_async_copy(dst_hbm.at[pl.ds(cb1, K)], odidx, osem_i).wait()
            pltpu.async_copy(xl_hbm.at[osidx], oxj, osem_g)
            pltpu.async_copy(xre_hbm.at[odidx], oxie, osem_g)

            def _edge(e2, _):
                _edge_compute(2 * e2, xj_v, xie_v, rbuf, att_rows, lane)
                _edge_compute(2 * e2 + 1, xj_v, xie_v, rbuf, att_rows, lane)
                return 0

            lax.fori_loop(0, K // 2, _edge, 0)
            pltpu.async_copy(rbuf, out_sh.at[didx_s], sem_s, add=True)
        return cnt

    cnt = lax.fori_loop(0, CHUNKS // 2, _pair, jnp.int32(0))

    pltpu.make_async_copy(xl_hbm.at[sidx0], xj0, sem_g0).wait()
    pltpu.make_async_copy(xre_hbm.at[didx0], xie0, sem_g0).wait()
    pltpu.make_async_copy(src_hbm.at[pl.ds(ebase, K)], sidx1, sem_i1).wait()
    pltpu.make_async_copy(dst_hbm.at[pl.ds(ebase, K)], didx1, sem_i1).wait()
    pltpu.make_async_copy(row_v, out_sh.at[didx_s0], sem_s0).wait()
    pltpu.make_async_copy(row_w, out_sh.at[didx_s1], sem_s1).wait()

    pltpu.sync_copy(filt_v, filt_hbm.at[pl.ds(wid * TE2, TE2)])
    @pl.when(wid == NW - 1)
    def _tail():
        pltpu.sync_copy(filt_v.at[pl.ds(TE, 2 * K2)],
                        filt_hbm.at[pl.ds(NW * TE2, 2 * K2)])

    c16_v[...] = jnp.full((16,), 1, jnp.int32) * cnt
    pltpu.sync_copy(c16_v, cnt_hbm.at[wid])

    plsc.subcore_barrier()
    wbase = sid * ROWS_PER_TILE
    pltpu.sync_copy(out_sh.at[pl.ds(wbase, ROWS_PER_TILE)],
                    out_hbm.at[cid, pl.ds(wbase, ROWS_PER_TILE)])


def _sc_body2(xl_hbm, xre_hbm, filt_hbm, cnt_hbm, att_hbm, out_hbm,
              pk0, pk1, sidx0, didx0, sidx1, didx1, xj0, xie0, xj1, xie1,
              row_v, att_v, c16_v, out_sh,
              sem_i0, sem_i1, sem_g0, sem_g1):
    cid = lax.axis_index("c")
    sid = lax.axis_index("s")
    wid = cid * 16 + sid

    pltpu.sync_copy(att_hbm, att_v)
    _zero_rows(row_v, K2)

    pltpu.sync_copy(row_v.at[pl.ds(0, NF // 16)],
                    out_sh.at[pl.ds(sid * (NF // 16), NF // 16)])
    plsc.subcore_barrier()

    pltpu.sync_copy(cnt_hbm.at[wid], c16_v)
    cnt = c16_v[pl.ds(0, 16)][0]
    npair = (cnt + (2 * K2 - 1)) // (2 * K2)

    lane = lax.broadcasted_iota(jnp.int32, (16,), 0)
    att_rows = [att_v[pl.ds(h * C, 16)] for h in range(H)]
    lbase = wid * TE2
    bufs = [(pk0, sidx0, didx0, xj0, xie0, sem_i0, sem_g0),
            (pk1, sidx1, didx1, xj1, xie1, sem_i1, sem_g1)]

    def _unpack(pk, sidx, didx):
        for g in range(K2 // 16):
            p = pk[pl.ds(g * 16, 16)]
            sidx[pl.ds(g * 16, 16)] = jnp.bitwise_and(p, PACKB - 1)
            didx[pl.ds(g * 16, 16)] = lax.shift_right_logical(p, 14)

    pltpu.async_copy(filt_hbm.at[pl.ds(lbase, K2)], pk0, sem_i0)
    pltpu.async_copy(filt_hbm.at[pl.ds(lbase + K2, K2)], pk1, sem_i1)
    pltpu.make_async_copy(filt_hbm.at[pl.ds(lbase, K2)], pk0, sem_i0).wait()
    _unpack(pk0, sidx0, didx0)
    pltpu.async_copy(xl_hbm.at[sidx0], xj0, sem_g0)
    pltpu.async_copy(xre_hbm.at[didx0], xie0, sem_g0)

    def _pair(kp, _):
        for sub in range(2):
            k = 2 * kp + sub
            pk, sidx, didx, xj_v, xie_v, sem_i, sem_g = bufs[sub]
            opk, osidx, odidx, oxj, oxie, osem_i, osem_g = bufs[1 - sub]
            pltpu.make_async_copy(xl_hbm.at[sidx], xj_v, sem_g).wait()
            pltpu.make_async_copy(xre_hbm.at[didx], xie_v, sem_g).wait()
            cb2 = lbase + (k + 2) * K2
            pltpu.async_copy(filt_hbm.at[pl.ds(cb2, K2)], pk, sem_i)
            pltpu.make_async_copy(filt_hbm.at[pl.ds(cb2, K2)], opk, osem_i).wait()
            _unpack(opk, osidx, odidx)
            pltpu.async_copy(xl_hbm.at[osidx], oxj, osem_g)
            pltpu.async_copy(xre_hbm.at[odidx], oxie, osem_g)

            def _edge(e2, _):
                _edge_compute(2 * e2, xj_v, xie_v, row_v, att_rows, lane)
                _edge_compute(2 * e2 + 1, xj_v, xie_v, row_v, att_rows, lane)
                return 0

            lax.fori_loop(0, K2 // 2, _edge, 0)
            pltpu.sync_copy(row_v, out_sh.at[didx], add=True)
        return 0

    lax.fori_loop(0, npair, _pair, 0)

    pltpu.make_async_copy(xl_hbm.at[sidx0], xj0, sem_g0).wait()
    pltpu.make_async_copy(xre_hbm.at[didx0], xie0, sem_g0).wait()
    pltpu.make_async_copy(filt_hbm.at[pl.ds(lbase, K2)], pk1, sem_i1).wait()

    plsc.subcore_barrier()
    wbase = sid * (NF // 16)
    pltpu.sync_copy(out_sh.at[pl.ds(wbase, NF // 16)],
                    out_hbm.at[cid, pl.ds(wbase, NF // 16)])


def _sc_mesh():
    return plsc.VectorSubcoreMesh(core_axis_name="c", subcore_axis_name="s",
                                  num_cores=2, num_subcores=16)


_SC_PARAMS = pltpu.CompilerParams(needs_layout_passes=False,
                                  use_tc_tiling_on_sc=False)


def _sc_edge_pass1(xl, xre, src, dst, attf):
    f = pl.kernel(
        _sc_body1,
        out_type=[
            jax.ShapeDtypeStruct((2, NP, OW), jnp.float32),
            jax.ShapeDtypeStruct((NW * TE2 + 2 * K2,), jnp.int32),
            jax.ShapeDtypeStruct((NW, 16), jnp.int32),
        ],
        mesh=_sc_mesh(),
        compiler_params=_SC_PARAMS,
        scratch_types=[
            pltpu.VMEM((K,), jnp.int32),
            pltpu.VMEM((K,), jnp.int32),
            pltpu.VMEM((K,), jnp.int32),
            pltpu.VMEM((K,), jnp.int32),
            pltpu.VMEM((K, HC), jnp.float32),
            pltpu.VMEM((K, OW), jnp.float32),
            pltpu.VMEM((K, HC), jnp.float32),
            pltpu.VMEM((K, OW), jnp.float32),
            pltpu.VMEM((K,), jnp.int32),
            pltpu.VMEM((K,), jnp.int32),
            pltpu.VMEM((K, OW), jnp.float32),
            pltpu.VMEM((K, OW), jnp.float32),
            pltpu.VMEM((HC,), jnp.float32),
            pltpu.VMEM((TE2,), jnp.int32),
            pltpu.VMEM((16,), jnp.int32),
            pltpu.VMEM_SHARED((NP, OW), jnp.float32),
            pltpu.SemaphoreType.DMA,
            pltpu.SemaphoreType.DMA,
            pltpu.SemaphoreType.DMA,
            pltpu.SemaphoreType.DMA,
            pltpu.SemaphoreType.DMA,
            pltpu.SemaphoreType.DMA,
        ],
    )
    return f(xl, xre, src, dst, attf)


def _sc_edge_pass2(xl, xre, filt, cnts, attf):
    f = pl.kernel(
        _sc_body2,
        out_type=jax.ShapeDtypeStruct((2, NF, OW), jnp.float32),
        mesh=_sc_mesh(),
        compiler_params=_SC_PARAMS,
        scratch_types=[
            pltpu.VMEM((K2,), jnp.int32),
            pltpu.VMEM((K2,), jnp.int32),
            pltpu.VMEM((K2,), jnp.int32),
            pltpu.VMEM((K2,), jnp.int32),
            pltpu.VMEM((K2,), jnp.int32),
            pltpu.VMEM((K2,), jnp.int32),
            pltpu.VMEM((K2, HC), jnp.float32),
            pltpu.VMEM((K2, OW), jnp.float32),
            pltpu.VMEM((K2, HC), jnp.float32),
            pltpu.VMEM((K2, OW), jnp.float32),
            pltpu.VMEM((K2, OW), jnp.float32),
            pltpu.VMEM((HC,), jnp.float32),
            pltpu.VMEM((16,), jnp.int32),
            pltpu.VMEM_SHARED((NF, OW), jnp.float32),
            pltpu.SemaphoreType.DMA,
            pltpu.SemaphoreType.DMA,
            pltpu.SemaphoreType.DMA,
            pltpu.SemaphoreType.DMA,
        ],
    )
    return f(xl, xre, filt, cnts, attf)


def kernel(x, edgeIds, W1l, b1l, W1r, b1r, att1, bias1,
           W2l, b2l, W2r, b2r, att2, bias2, Wc, bc):
    f32 = jnp.float32
    xp = jnp.zeros((NP, D), f32).at[:N].set(x)
    loop = jnp.arange(N, dtype=jnp.int32)

    halfw = NW // 2 * TE
    hpad = jnp.full((halfw - E // 2 - N // 2,), DUMMY, jnp.int32)
    tpad = jnp.full((halfw - E // 2 - N // 2 + EXTRA,), DUMMY, jnp.int32)

    def _stream(a):
        a = a.astype(jnp.int32)
        return jnp.concatenate([a[:E // 2], loop[:N // 2], hpad,
                                a[E // 2:], loop[N // 2:], tpad])

    src = _stream(edgeIds[0])
    dst = _stream(edgeIds[1])

    m = (jnp.arange(HC)[:, None] // C == jnp.arange(C)[None, :]).astype(f32)
    mexp = m.T
    att1f = att1.reshape(1, HC)
    att2f = att2.reshape(1, HC)

    xl1, xre1 = _tc1(xp, W1l, b1l.reshape(1, HC), W1r, b1r.reshape(1, HC),
                     att1f, m)
    parts1, filt, cnts = _sc_edge_pass1(xl1, xre1, src, dst, att1f.reshape(HC))
    xl2, xre2 = _tc2(parts1, mexp, bias1.reshape(1, HC), W2l,
                     b2l.reshape(1, HC), W2r, b2r.reshape(1, HC), att2f, m)
    parts2 = _sc_edge_pass2(xl2, xre2, filt, cnts, att2f.reshape(HC))
    out = _tc3(parts2, mexp, bias2.reshape(1, HC), Wc, bc.reshape(1, NCLS))
    return out


# speedup vs baseline: 1.3049x; 1.1890x over previous
import functools

import jax
import jax.numpy as jnp
from jax import lax
from jax.experimental import pallas as pl
from jax.experimental.pallas import tpu as pltpu
from jax.experimental.pallas import tpu_sc as plsc

N = 10000
D = 128
H = 8
C = 16
HC = H * C
NCLS = 16

NP = 10240
DUMMY = N
E = 320000
ESL = E + N
NW = 32
K = 32
TE = 10368
EPAD = NW * TE
CHUNKS = TE // K
EXTRA = 2 * K
ROWS_PER_TILE = NP // 16
OW = 144


def _lrelu(v):
    return jnp.maximum(v, 0.2 * v)


def _dot(a, b):
    return jax.lax.dot_general(a, b, (((1,), (0,)), ((), ())),
                               precision=jax.lax.Precision.HIGHEST,
                               preferred_element_type=jnp.float32)


def _tc1_body(x_ref, wl_ref, bl_ref, wr_ref, br_ref, att_ref, m_ref,
              xl_ref, xre_ref):
    x = x_ref[...]
    xl = _dot(x, wl_ref[...]) + bl_ref[...]
    xr = _dot(x, wr_ref[...]) + br_ref[...]
    t = att_ref[...] * _lrelu(xl + xr)
    xl_ref[...] = xl
    xre_ref[...] = jnp.concatenate([xr, _dot(t, m_ref[...])], axis=-1)


def _tc1(x, wl, bl, wr, br, attf, m):
    bn = 512
    grid = (NP // bn,)
    return pl.pallas_call(
        _tc1_body,
        grid=grid,
        in_specs=[
            pl.BlockSpec((bn, D), lambda i: (i, 0)),
            pl.BlockSpec((D, HC), lambda i: (0, 0)),
            pl.BlockSpec((1, HC), lambda i: (0, 0)),
            pl.BlockSpec((D, HC), lambda i: (0, 0)),
            pl.BlockSpec((1, HC), lambda i: (0, 0)),
            pl.BlockSpec((1, HC), lambda i: (0, 0)),
            pl.BlockSpec((HC, C), lambda i: (0, 0)),
        ],
        out_specs=[
            pl.BlockSpec((bn, HC), lambda i: (i, 0)),
            pl.BlockSpec((bn, OW), lambda i: (i, 0)),
        ],
        out_shape=[
            jax.ShapeDtypeStruct((NP, HC), jnp.float32),
            jax.ShapeDtypeStruct((NP, OW), jnp.float32),
        ],
    )(x, wl, bl, wr, br, attf, m)


def _tc2_body(pa_ref, pb_ref, mexp_ref, b_prev_ref, wl_ref, bl_ref,
              wr_ref, br_ref, att_ref, m_ref, xl_ref, xre_ref):
    p = pa_ref[0] + pb_ref[0]
    out_un = p[:, :HC]
    s_exp = _dot(p[:, HC:OW], mexp_ref[...])
    h = jnp.maximum(out_un / s_exp + b_prev_ref[...], 0.0)
    xl = _dot(h, wl_ref[...]) + bl_ref[...]
    xr = _dot(h, wr_ref[...]) + br_ref[...]
    t = att_ref[...] * _lrelu(xl + xr)
    xl_ref[...] = xl
    xre_ref[...] = jnp.concatenate([xr, _dot(t, m_ref[...])], axis=-1)


def _tc2(parts, mexp, b_prev, wl, bl, wr, br, attf, m):
    bn = 512
    grid = (NP // bn,)
    return pl.pallas_call(
        _tc2_body,
        grid=grid,
        in_specs=[
            pl.BlockSpec((1, bn, OW), lambda i: (0, i, 0)),
            pl.BlockSpec((1, bn, OW), lambda i: (1, i, 0)),
            pl.BlockSpec((C, HC), lambda i: (0, 0)),
            pl.BlockSpec((1, HC), lambda i: (0, 0)),
            pl.BlockSpec((D, HC), lambda i: (0, 0)),
            pl.BlockSpec((1, HC), lambda i: (0, 0)),
            pl.BlockSpec((D, HC), lambda i: (0, 0)),
            pl.BlockSpec((1, HC), lambda i: (0, 0)),
            pl.BlockSpec((1, HC), lambda i: (0, 0)),
            pl.BlockSpec((HC, C), lambda i: (0, 0)),
        ],
        out_specs=[
            pl.BlockSpec((bn, HC), lambda i: (i, 0)),
            pl.BlockSpec((bn, OW), lambda i: (i, 0)),
        ],
        out_shape=[
            jax.ShapeDtypeStruct((NP, HC), jnp.float32),
            jax.ShapeDtypeStruct((NP, OW), jnp.float32),
        ],
    )(parts, parts, mexp, b_prev, wl, bl, wr, br, attf, m)


NF = 256
FDUMMY = 255
PACKB = 16384
K2 = 32
TE2 = TE + 2 * K2


def _tc3_body(pa_ref, pb_ref, mexp_ref, b_prev_ref, wc_ref, bc_ref, out_ref):
    p = pa_ref[0] + pb_ref[0]
    out_un = p[:, :HC]
    s_exp = _dot(p[:, HC:OW], mexp_ref[...])
    h = out_un / s_exp + b_prev_ref[...]
    r = lax.broadcasted_iota(jnp.int32, (256, HC), 0)
    vis = jnp.sum(jnp.where(r < 100, h, 0.0), axis=0, keepdims=True) / 100.0
    aud = jnp.sum(jnp.where((r >= 100) & (r < 200), h, 0.0), axis=0,
                  keepdims=True) / 100.0
    tx = jnp.sum(jnp.where(r == 200, h, 0.0), axis=0, keepdims=True)
    avg = (vis + aud + tx) / 3.0
    out_ref[...] = _dot(avg, wc_ref[...]) + bc_ref[...]


def _tc3(parts, mexp, b_prev, wc, bc):
    return pl.pallas_call(
        _tc3_body,
        grid=(1,),
        in_specs=[
            pl.BlockSpec((1, NF, OW), lambda i: (0, 0, 0)),
            pl.BlockSpec((1, NF, OW), lambda i: (1, 0, 0)),
            pl.BlockSpec((C, HC), lambda i: (0, 0)),
            pl.BlockSpec((1, HC), lambda i: (0, 0)),
            pl.BlockSpec((HC, NCLS), lambda i: (0, 0)),
            pl.BlockSpec((1, NCLS), lambda i: (0, 0)),
        ],
        out_specs=pl.BlockSpec((1, NCLS), lambda i: (0, 0)),
        out_shape=jax.ShapeDtypeStruct((1, NCLS), jnp.float32),
    )(parts, parts, mexp, b_prev, wc, bc)


def _edge_compute(e, xj_v, xie_v, row_v, att_rows, lane):
    a_vec = jnp.zeros((16,), jnp.float32)
    xjs = []
    for h in range(H):
        xj = xj_v[e, pl.ds(h * C, 16)]
        xi = xie_v[e, pl.ds(h * C, 16)]
        t = att_rows[h] * _lrelu(xi + xj)
        a_vec = jnp.where(lane == h, jnp.sum(t), a_vec)
        xjs.append(xj)
    shr = xie_v[e, pl.ds(HC, 16)]
    ex_vec = jnp.exp(a_vec - shr)
    row_v[e, pl.ds(HC, 16)] = ex_vec
    for h in range(H):
        row_v[e, pl.ds(h * C, 16)] = xjs[h] * ex_vec[h]


def _zero_rows(row_v, nrows):
    zv = jnp.zeros((16,), jnp.float32)

    def _zero_row(r, _):
        for j in range(OW // 16):
            row_v[r, pl.ds(j * 16, 16)] = zv
        return 0

    lax.fori_loop(0, nrows, _zero_row, 0)


def _sc_body1(xl_hbm, xre_hbm, src_hbm, dst_hbm, att_hbm,
              out_hbm, filt_hbm, cnt_hbm,
              sidx0, didx0, sidx1, didx1, xj0, xie0, xj1, xie1,
              didx_s0, didx_s1, row_v, row_w, att_v, filt_v, c16_v, out_sh,
              sem_i0, sem_i1, sem_g0, sem_g1, sem_s0, sem_s1):
    cid = lax.axis_index("c")
    sid = lax.axis_index("s")
    wid = cid * 16 + sid

    pltpu.sync_copy(att_hbm, att_v)
    _zero_rows(row_v, K)
    _zero_rows(row_w, K)

    dlane = lax.broadcasted_iota(jnp.int32, (16,), 0)

    def _fill(i, _):
        drow = 201 + lax.rem(i, 3) * 16 + dlane
        filt_v[pl.ds(i * 16, 16)] = drow * PACKB + DUMMY
        return 0

    lax.fori_loop(0, TE2 // 16, _fill, 0)

    zbase = sid * ROWS_PER_TILE
    for i in range(ROWS_PER_TILE // K):
        pltpu.sync_copy(row_v, out_sh.at[pl.ds(zbase + i * K, K)])
    plsc.subcore_barrier()

    ebase = wid * TE
    lane = lax.broadcasted_iota(jnp.int32, (16,), 0)
    att_rows = [att_v[pl.ds(h * C, 16)] for h in range(H)]
    bufs = [(sidx0, didx0, xj0, xie0, sem_i0, sem_g0, didx_s0, row_v, sem_s0),
            (sidx1, didx1, xj1, xie1, sem_i1, sem_g1, didx_s1, row_w, sem_s1)]

    pltpu.async_copy(src_hbm.at[pl.ds(ebase, K)], sidx0, sem_i0)
    pltpu.async_copy(dst_hbm.at[pl.ds(ebase, K)], didx0, sem_i0)
    pltpu.async_copy(src_hbm.at[pl.ds(ebase + K, K)], sidx1, sem_i1)
    pltpu.async_copy(dst_hbm.at[pl.ds(ebase + K, K)], didx1, sem_i1)
    pltpu.make_async_copy(src_hbm.at[pl.ds(ebase, K)], sidx0, sem_i0).wait()
    pltpu.make_async_copy(dst_hbm.at[pl.ds(ebase, K)], didx0, sem_i0).wait()
    pltpu.async_copy(xl_hbm.at[sidx0], xj0, sem_g0)
    pltpu.async_copy(xre_hbm.at[didx0], xie0, sem_g0)

    def _pair(k2, cnt):
        for sub in range(2):
            k = 2 * k2 + sub
            sidx, didx, xj_v, xie_v, sem_i, sem_g, didx_s, rbuf, sem_s = bufs[sub]
            osidx, odidx, oxj, oxie, osem_i, osem_g, _, _, _ = bufs[1 - sub]
            pltpu.make_async_copy(xl_hbm.at[sidx], xj_v, sem_g).wait()
            pltpu.make_async_copy(xre_hbm.at[didx], xie_v, sem_g).wait()

            @pl.when(k2 > 0)
            def _drain():
                pltpu.make_async_copy(rbuf, out_sh.at[didx_s], sem_s).wait()

            for g in range(K // 16):
                sv = sidx[pl.ds(g * 16, 16)]
                dv = didx[pl.ds(g * 16, 16)]
                didx_s[pl.ds(g * 16, 16)] = dv
                mask = dv <= 200
                mi = mask.astype(jnp.int32)
                pos = cnt + plsc.cumsum(mi) - 1
                plsc.store_scatter(filt_v, [pos], dv * PACKB + sv, mask=mask)
                cnt = cnt + plsc.all_reduce_population_count(mask)[0]
            cb2 = ebase + (k + 2) * K
            pltpu.async_copy(src_hbm.at[pl.ds(cb2, K)], sidx, sem_i)
            pltpu.async_copy(dst_hbm.at[pl.ds(cb2, K)], didx, sem_i)
            cb1 = ebase + (k + 1) * K
            pltpu.make_async_copy(src_hbm.at[pl.ds(cb1, K)], osidx, osem_i).wait()
            pltpu.make_async_copy(dst_hbm.at[pl.ds(cb1, K)], odidx, osem_i).wait()
            pltpu.async_copy(xl_hbm.at[osidx], oxj, osem_g)
            pltpu.async_copy(xre_hbm.at[odidx], oxie, osem_g)

            def _edge(e2, _):
                _edge_compute(2 * e2, xj_v, xie_v, rbuf, att_rows, lane)
                _edge_compute(2 * e2 + 1, xj_v, xie_v, rbuf, att_rows, lane)
                return 0

            lax.fori_loop(0, K // 2, _edge, 0)
            pltpu.async_copy(rbuf, out_sh.at[didx_s], sem_s, add=True)
        return cnt

    cnt = lax.fori_loop(0, CHUNKS // 2, _pair, jnp.int32(0))

    pltpu.make_async_copy(xl_hbm.at[sidx0], xj0, sem_g0).wait()
    pltpu.make_async_copy(xre_hbm.at[didx0], xie0, sem_g0).wait()
    pltpu.make_async_copy(src_hbm.at[pl.ds(ebase, K)], sidx1, sem_i1).wait()
    pltpu.make_async_copy(dst_hbm.at[pl.ds(ebase, K)], didx1, sem_i1).wait()
    pltpu.make_async_copy(row_v, out_sh.at[didx_s0], sem_s0).wait()
    pltpu.make_async_copy(row_w, out_sh.at[didx_s1], sem_s1).wait()

    pltpu.sync_copy(filt_v, filt_hbm.at[pl.ds(wid * TE2, TE2)])
    @pl.when(wid == NW - 1)
    def _tail():
        pltpu.sync_copy(filt_v.at[pl.ds(TE, 2 * K2)],
                        filt_hbm.at[pl.ds(NW * TE2, 2 * K2)])

    c16_v[...] = jnp.full((16,), 1, jnp.int32) * cnt
    pltpu.sync_copy(c16_v, cnt_hbm.at[wid])

    plsc.subcore_barrier()
    wbase = sid * ROWS_PER_TILE
    pltpu.sync_copy(out_sh.at[pl.ds(wbase, ROWS_PER_TILE)],
                    out_hbm.at[cid, pl.ds(wbase, ROWS_PER_TILE)])


def _sc_body2(xl_hbm, xre_hbm, filt_hbm, cnt_hbm, att_hbm, out_hbm,
              pk0, pk1, sidx0, didx0, sidx1, didx1, xj0, xie0, xj1, xie1,
              row_v, att_v, c16_v, out_sh,
              sem_i0, sem_i1, sem_g0, sem_g1):
    cid = lax.axis_index("c")
    sid = lax.axis_index("s")
    wid = cid * 16 + sid

    pltpu.sync_copy(att_hbm, att_v)
    _zero_rows(row_v, K2)

    pltpu.sync_copy(row_v.at[pl.ds(0, NF // 16)],
                    out_sh.at[pl.ds(sid * (NF // 16), NF // 16)])
    plsc.subcore_barrier()

    pltpu.sync_copy(cnt_hbm.at[wid], c16_v)
    cnt = c16_v[pl.ds(0, 16)][0]
    npair = (cnt + (2 * K2 - 1)) // (2 * K2)

    lane = lax.broadcasted_iota(jnp.int32, (16,), 0)
    att_rows = [att_v[pl.ds(h * C, 16)] for h in range(H)]
    lbase = wid * TE2
    bufs = [(pk0, sidx0, didx0, xj0, xie0, sem_i0, sem_g0),
            (pk1, sidx1, didx1, xj1, xie1, sem_i1, sem_g1)]

    def _unpack(pk, sidx, didx):
        for g in range(K2 // 16):
            p = pk[pl.ds(g * 16, 16)]
            sidx[pl.ds(g * 16, 16)] = jnp.bitwise_and(p, PACKB - 1)
            didx[pl.ds(g * 16, 16)] = lax.shift_right_logical(p, 14)

    pltpu.async_copy(filt_hbm.at[pl.ds(lbase, K2)], pk0, sem_i0)
    pltpu.async_copy(filt_hbm.at[pl.ds(lbase + K2, K2)], pk1, sem_i1)
    pltpu.make_async_copy(filt_hbm.at[pl.ds(lbase, K2)], pk0, sem_i0).wait()
    _unpack(pk0, sidx0, didx0)
    pltpu.async_copy(xl_hbm.at[sidx0], xj0, sem_g0)
    pltpu.async_copy(xre_hbm.at[didx0], xie0, sem_g0)

    def _pair(kp, _):
        for sub in range(2):
            k = 2 * kp + sub
            pk, sidx, didx, xj_v, xie_v, sem_i, sem_g = bufs[sub]
            opk, osidx, odidx, oxj, oxie, osem_i, osem_g = bufs[1 - sub]
            pltpu.make_async_copy(xl_hbm.at[sidx], xj_v, sem_g).wait()
            pltpu.make_async_copy(xre_hbm.at[didx], xie_v, sem_g).wait()
            cb2 = lbase + (k + 2) * K2
            pltpu.async_copy(filt_hbm.at[pl.ds(cb2, K2)], pk, sem_i)
            pltpu.make_async_copy(filt_hbm.at[pl.ds(cb2, K2)], opk, osem_i).wait()
            _unpack(opk, osidx, odidx)
            pltpu.async_copy(xl_hbm.at[osidx], oxj, osem_g)
            pltpu.async_copy(xre_hbm.at[odidx], oxie, osem_g)

            def _edge(e2, _):
                _edge_compute(2 * e2, xj_v, xie_v, row_v, att_rows, lane)
                _edge_compute(2 * e2 + 1, xj_v, xie_v, row_v, att_rows, lane)
                return 0

            lax.fori_loop(0, K2 // 2, _edge, 0)
            pltpu.sync_copy(row_v, out_sh.at[didx], add=True)
        return 0

    lax.fori_loop(0, npair, _pair, 0)

    pltpu.make_async_copy(xl_hbm.at[sidx0], xj0, sem_g0).wait()
    pltpu.make_async_copy(xre_hbm.at[didx0], xie0, sem_g0).wait()
    pltpu.make_async_copy(filt_hbm.at[pl.ds(lbase, K2)], pk1, sem_i1).wait()

    plsc.subcore_barrier()
    wbase = sid * (NF // 16)
    pltpu.sync_copy(out_sh.at[pl.ds(wbase, NF // 16)],
                    out_hbm.at[cid, pl.ds(wbase, NF // 16)])


def _sc_mesh():
    return plsc.VectorSubcoreMesh(core_axis_name="c", subcore_axis_name="s",
                                  num_cores=2, num_subcores=16)


_SC_PARAMS = pltpu.CompilerParams(needs_layout_passes=False,
                                  use_tc_tiling_on_sc=False)


def _sc_edge_pass1(xl, xre, src, dst, attf):
    f = pl.kernel(
        _sc_body1,
        out_type=[
            jax.ShapeDtypeStruct((2, NP, OW), jnp.float32),
            jax.ShapeDtypeStruct((NW * TE2 + 2 * K2,), jnp.int32),
            jax.ShapeDtypeStruct((NW, 16), jnp.int32),
        ],
        mesh=_sc_mesh(),
        compiler_params=_SC_PARAMS,
        scratch_types=[
            pltpu.VMEM((K,), jnp.int32),
            pltpu.VMEM((K,), jnp.int32),
            pltpu.VMEM((K,), jnp.int32),
            pltpu.VMEM((K,), jnp.int32),
            pltpu.VMEM((K, HC), jnp.float32),
            pltpu.VMEM((K, OW), jnp.float32),
            pltpu.VMEM((K, HC), jnp.float32),
            pltpu.VMEM((K, OW), jnp.float32),
            pltpu.VMEM((K,), jnp.int32),
            pltpu.VMEM((K,), jnp.int32),
            pltpu.VMEM((K, OW), jnp.float32),
            pltpu.VMEM((K, OW), jnp.float32),
            pltpu.VMEM((HC,), jnp.float32),
            pltpu.VMEM((TE2,), jnp.int32),
            pltpu.VMEM((16,), jnp.int32),
            pltpu.VMEM_SHARED((NP, OW), jnp.float32),
            pltpu.SemaphoreType.DMA,
            pltpu.SemaphoreType.DMA,
            pltpu.SemaphoreType.DMA,
            pltpu.SemaphoreType.DMA,
            pltpu.SemaphoreType.DMA,
            pltpu.SemaphoreType.DMA,
        ],
    )
    return f(xl, xre, src, dst, attf)


def _sc_edge_pass2(xl, xre, filt, cnts, attf):
    f = pl.kernel(
        _sc_body2,
        out_type=jax.ShapeDtypeStruct((2, NF, OW), jnp.float32),
        mesh=_sc_mesh(),
        compiler_params=_SC_PARAMS,
        scratch_types=[
            pltpu.VMEM((K2,), jnp.int32),
            pltpu.VMEM((K2,), jnp.int32),
            pltpu.VMEM((K2,), jnp.int32),
            pltpu.VMEM((K2,), jnp.int32),
            pltpu.VMEM((K2,), jnp.int32),
            pltpu.VMEM((K2,), jnp.int32),
            pltpu.VMEM((K2, HC), jnp.float32),
            pltpu.VMEM((K2, OW), jnp.float32),
            pltpu.VMEM((K2, HC), jnp.float32),
            pltpu.VMEM((K2, OW), jnp.float32),
            pltpu.VMEM((K2, OW), jnp.float32),
            pltpu.VMEM((HC,), jnp.float32),
            pltpu.VMEM((16,), jnp.int32),
            pltpu.VMEM_SHARED((NF, OW), jnp.float32),
            pltpu.SemaphoreType.DMA,
            pltpu.SemaphoreType.DMA,
            pltpu.SemaphoreType.DMA,
            pltpu.SemaphoreType.DMA,
        ],
    )
    return f(xl, xre, filt, cnts, attf)


def kernel(x, edgeIds, W1l, b1l, W1r, b1r, att1, bias1,
           W2l, b2l, W2r, b2r, att2, bias2, Wc, bc):
    f32 = jnp.float32
    xp = jnp.zeros((NP, D), f32).at[:N].set(x)
    loop = jnp.arange(N, dtype=jnp.int32)

    halfw = NW // 2 * TE
    hpad = jnp.full((halfw - E // 2 - N // 2,), DUMMY, jnp.int32)
    tpad = jnp.full((halfw - E // 2 - N // 2 + EXTRA,), DUMMY, jnp.int32)

    def _stream(a):
        a = a.astype(jnp.int32)
        return jnp.concatenate([a[:E // 2], loop[:N // 2], hpad,
                                a[E // 2:], loop[N // 2:], tpad])

    src = _stream(edgeIds[0])
    dst = _stream(edgeIds[1])

    m = (jnp.arange(HC)[:, None] // C == jnp.arange(C)[None, :]).astype(f32)
    mexp = m.T
    att1f = att1.reshape(1, HC)
    att2f = att2.reshape(1, HC)

    xl1, xre1 = _tc1(xp, W1l, b1l.reshape(1, HC), W1r, b1r.reshape(1, HC),
                     att1f, m)
    parts1, filt, cnts = _sc_edge_pass1(xl1, xre1, src, dst, att1f.reshape(HC))
    xl2, xre2 = _tc2(parts1, mexp, bias1.reshape(1, HC), W2l,
                     b2l.reshape(1, HC), W2r, b2r.reshape(1, HC), att2f, m)
    parts2 = _sc_edge_pass2(xl2, xre2, filt, cnts, att2f.reshape(HC))
    out = _tc3(parts2, mexp, bias2.reshape(1, HC), Wc, bc.reshape(1, NCLS))
    return out
